# initial kernel scaffold (unmeasured)
import jax
import jax.numpy as jnp
from jax import lax
from jax.experimental import pallas as pl
from jax.experimental.pallas import tpu as pltpu


def kernel(
    x,
):
    def body(*refs):
        pass

    out_shape = jax.ShapeDtypeStruct(..., jnp.float32)
    return pl.pallas_call(body, out_shape=out_shape)(...)


# baseline (device time: 315655 ns/iter reference)
import jax
import jax.numpy as jnp
from jax import lax
from jax.experimental import pallas as pl
from jax.experimental.pallas import tpu as pltpu

N_DEV = 4
M = 4096
N = 2048
CHUNK = M // N_DEV
N_STEPS = 2 * (N_DEV - 1)


def kernel(x):

    def body(x_ref, out_ref, comm_ref, send_sems, recv_sems):
        my = lax.axis_index("i")
        right = lax.rem(my + 1, N_DEV)
        left = lax.rem(my + N_DEV - 1, N_DEV)

        barrier_sem = pltpu.get_barrier_semaphore()
        for nbr in (left, right):
            pl.semaphore_signal(
                barrier_sem, inc=1,
                device_id=(nbr,), device_id_type=pl.DeviceIdType.MESH,
            )
        pl.semaphore_wait(barrier_sem, 2)

        out_ref[...] = x_ref[0].astype(jnp.bfloat16)

        for s in range(N_DEV - 1):
            send_c = lax.rem(my - s + N_DEV, N_DEV)
            recv_c = lax.rem(my - s - 1 + N_DEV, N_DEV)
            rdma = pltpu.make_async_remote_copy(
                src_ref=out_ref.at[pl.ds(send_c * CHUNK, CHUNK), :],
                dst_ref=comm_ref.at[s],
                send_sem=send_sems.at[s],
                recv_sem=recv_sems.at[s],
                device_id=(right,),
                device_id_type=pl.DeviceIdType.MESH,
            )
            rdma.start()
            rdma.wait()
            off = recv_c * CHUNK
            out_ref[pl.ds(off, CHUNK), :] = (
                out_ref[pl.ds(off, CHUNK), :] + comm_ref[s]
            )

        for t in range(N_DEV - 1):
            s = N_DEV - 1 + t
            send_c = lax.rem(my + 1 - t + N_DEV, N_DEV)
            rdma = pltpu.make_async_remote_copy(
                src_ref=out_ref.at[pl.ds(send_c * CHUNK, CHUNK), :],
                dst_ref=out_ref.at[pl.ds(send_c * CHUNK, CHUNK), :],
                send_sem=send_sems.at[s],
                recv_sem=recv_sems.at[s],
                device_id=(right,),
                device_id_type=pl.DeviceIdType.MESH,
            )
            rdma.start()
            rdma.wait()

    return pl.pallas_call(
        body,
        out_shape=jax.ShapeDtypeStruct((M, N), jnp.bfloat16),
        in_specs=[pl.BlockSpec(memory_space=pltpu.VMEM)],
        out_specs=pl.BlockSpec(memory_space=pltpu.VMEM),
        scratch_shapes=[
            pltpu.VMEM((N_DEV - 1, CHUNK, N), jnp.bfloat16),
            pltpu.SemaphoreType.DMA((N_STEPS,)),
            pltpu.SemaphoreType.DMA((N_STEPS,)),
        ],
        compiler_params=pltpu.CompilerParams(
            collective_id=0,
            vmem_limit_bytes=100 * 1024 * 1024,
        ),
    )(x)


# device time: 180904 ns/iter; 1.7449x vs baseline; 1.7449x over previous
import jax
import jax.numpy as jnp
from jax import lax
from jax.experimental import pallas as pl
from jax.experimental.pallas import tpu as pltpu

N_DEV = 4
M = 4096
N = 2048
CHUNK = M // N_DEV
HALF = N // 2
N_STEPS = 2 * (N_DEV - 1)


def kernel(x):

    def body(x_ref, out_ref, comm_r, comm_l,
             send_r, recv_r, send_l, recv_l):
        my = lax.axis_index("i")
        right = lax.rem(my + 1, N_DEV)
        left = lax.rem(my + N_DEV - 1, N_DEV)

        barrier_sem = pltpu.get_barrier_semaphore()
        for nbr in (left, right):
            pl.semaphore_signal(
                barrier_sem, inc=1,
                device_id=(nbr,), device_id_type=pl.DeviceIdType.MESH,
            )
        pl.semaphore_wait(barrier_sem, 2)

        out_ref[...] = x_ref[0].astype(jnp.bfloat16)

        def rs_rdma(s, send_c, dst, sems_s, sems_r, col0, dev):
            return pltpu.make_async_remote_copy(
                src_ref=out_ref.at[pl.ds(send_c * CHUNK, CHUNK),
                                   pl.ds(col0, HALF)],
                dst_ref=dst.at[s],
                send_sem=sems_s.at[s],
                recv_sem=sems_r.at[s],
                device_id=(dev,),
                device_id_type=pl.DeviceIdType.MESH,
            )

        for s in range(N_DEV - 1):
            send_cr = lax.rem(my - s + N_DEV, N_DEV)
            recv_cr = lax.rem(my - s - 1 + N_DEV, N_DEV)
            send_cl = lax.rem(my + s, N_DEV)
            recv_cl = lax.rem(my + s + 1, N_DEV)
            rr = rs_rdma(s, send_cr, comm_r, send_r, recv_r, 0, right)
            rl = rs_rdma(s, send_cl, comm_l, send_l, recv_l, HALF, left)
            rr.start()
            rl.start()
            rr.wait_recv()
            offr = recv_cr * CHUNK
            out_ref[pl.ds(offr, CHUNK), pl.ds(0, HALF)] = (
                out_ref[pl.ds(offr, CHUNK), pl.ds(0, HALF)] + comm_r[s]
            )
            rl.wait_recv()
            offl = recv_cl * CHUNK
            out_ref[pl.ds(offl, CHUNK), pl.ds(HALF, HALF)] = (
                out_ref[pl.ds(offl, CHUNK), pl.ds(HALF, HALF)] + comm_l[s]
            )
            rr.wait_send()
            rl.wait_send()

        def ag_rdma(s, send_c, sems_s, sems_r, col0, dev):
            sl = (pl.ds(send_c * CHUNK, CHUNK), pl.ds(col0, HALF))
            return pltpu.make_async_remote_copy(
                src_ref=out_ref.at[sl],
                dst_ref=out_ref.at[sl],
                send_sem=sems_s.at[s],
                recv_sem=sems_r.at[s],
                device_id=(dev,),
                device_id_type=pl.DeviceIdType.MESH,
            )

        for t in range(N_DEV - 1):
            s = N_DEV - 1 + t
            send_cr = lax.rem(my + 1 - t + N_DEV, N_DEV)
            send_cl = lax.rem(my - 1 + t + N_DEV, N_DEV)
            rr = ag_rdma(s, send_cr, send_r, recv_r, 0, right)
            rl = ag_rdma(s, send_cl, send_l, recv_l, HALF, left)
            rr.start()
            rl.start()
            rr.wait()
            rl.wait()

    return pl.pallas_call(
        body,
        out_shape=jax.ShapeDtypeStruct((M, N), jnp.bfloat16),
        in_specs=[pl.BlockSpec(memory_space=pltpu.VMEM)],
        out_specs=pl.BlockSpec(memory_space=pltpu.VMEM),
        scratch_shapes=[
            pltpu.VMEM((N_DEV - 1, CHUNK, HALF), jnp.bfloat16),
            pltpu.VMEM((N_DEV - 1, CHUNK, HALF), jnp.bfloat16),
            pltpu.SemaphoreType.DMA((N_STEPS,)),
            pltpu.SemaphoreType.DMA((N_STEPS,)),
            pltpu.SemaphoreType.DMA((N_STEPS,)),
            pltpu.SemaphoreType.DMA((N_STEPS,)),
        ],
        compiler_params=pltpu.CompilerParams(
            collective_id=0,
            vmem_limit_bytes=100 * 1024 * 1024,
        ),
    )(x)


# device time: 170504 ns/iter; 1.8513x vs baseline; 1.0610x over previous
import jax
import jax.numpy as jnp
from jax import lax
from jax.experimental import pallas as pl
from jax.experimental.pallas import tpu as pltpu

N_DEV = 4
M = 4096
N = 2048
CHUNK = M // N_DEV
HALF = N // 2
K = 2
Q = HALF // K
N_STEPS = 2 * (N_DEV - 1)


def kernel(x):

    def body(x_ref, out_ref, comm_r, comm_l,
             send_r, recv_r, send_l, recv_l):
        my = lax.axis_index("i")
        right = lax.rem(my + 1, N_DEV)
        left = lax.rem(my + N_DEV - 1, N_DEV)

        barrier_sem = pltpu.get_barrier_semaphore()
        for nbr in (left, right):
            pl.semaphore_signal(
                barrier_sem, inc=1,
                device_id=(nbr,), device_id_type=pl.DeviceIdType.MESH,
            )
        pl.semaphore_wait(barrier_sem, 2)

        def ring(d):
            return (
                (comm_r, send_r, recv_r, right) if d == 0
                else (comm_l, send_l, recv_l, left)
            )

        def rs_desc(s, k, chunk, d):
            comm, ssem, rsem, dev = ring(d)
            return pltpu.make_async_remote_copy(
                src_ref=out_ref.at[pl.ds(chunk * CHUNK, CHUNK),
                                   pl.ds(d * HALF + k * Q, Q)],
                dst_ref=comm.at[s, :, pl.ds(k * Q, Q)],
                send_sem=ssem.at[s, k],
                recv_sem=rsem.at[s, k],
                device_id=(dev,),
                device_id_type=pl.DeviceIdType.MESH,
            )

        def ag_desc(t, k, chunk, d):
            comm, ssem, rsem, dev = ring(d)
            sl = (pl.ds(chunk * CHUNK, CHUNK), pl.ds(d * HALF + k * Q, Q))
            return pltpu.make_async_remote_copy(
                src_ref=out_ref.at[sl],
                dst_ref=out_ref.at[sl],
                send_sem=ssem.at[N_DEV - 1 + t, k],
                recv_sem=rsem.at[N_DEV - 1 + t, k],
                device_id=(dev,),
                device_id_type=pl.DeviceIdType.MESH,
            )

        out_ref[pl.ds(my * CHUNK, CHUNK), :] = (
            x_ref[0, pl.ds(my * CHUNK, CHUNK), :].astype(jnp.bfloat16)
        )
        for k in range(K):
            for d in (0, 1):
                rs_desc(0, k, my, d).start()

        for s in range(N_DEV - 1):
            for k in range(K):
                for d in (0, 1):
                    if d == 0:
                        recv_c = lax.rem(my - s - 1 + N_DEV, N_DEV)
                    else:
                        recv_c = lax.rem(my + s + 1, N_DEV)
                    comm = ring(d)[0]
                    rs_desc(s, k, 0, d).wait_recv()
                    rows = pl.ds(recv_c * CHUNK, CHUNK)
                    cols = pl.ds(d * HALF + k * Q, Q)
                    out_ref[rows, cols] = (
                        x_ref[0, rows, cols].astype(jnp.bfloat16)
                        + comm[s, :, pl.ds(k * Q, Q)]
                    )
                    if s < N_DEV - 2:
                        rs_desc(s + 1, k, recv_c, d).start()
                    else:
                        ag_desc(0, k, recv_c, d).start()

        for t in range(N_DEV - 1):
            for k in range(K):
                for d in (0, 1):
                    if d == 0:
                        recv_c = lax.rem(my - t + N_DEV, N_DEV)
                    else:
                        recv_c = lax.rem(my + t, N_DEV)
                    ag_desc(t, k, recv_c, d).wait_recv()
                    if t < N_DEV - 2:
                        ag_desc(t + 1, k, recv_c, d).start()

        for s in range(N_DEV - 1):
            for k in range(K):
                for d in (0, 1):
                    rs_desc(s, k, 0, d).wait_send()
        for t in range(N_DEV - 1):
            for k in range(K):
                for d in (0, 1):
                    ag_desc(t, k, 0, d).wait_send()

    return pl.pallas_call(
        body,
        out_shape=jax.ShapeDtypeStruct((M, N), jnp.bfloat16),
        in_specs=[pl.BlockSpec(memory_space=pltpu.VMEM)],
        out_specs=pl.BlockSpec(memory_space=pltpu.VMEM),
        scratch_shapes=[
            pltpu.VMEM((N_DEV - 1, CHUNK, HALF), jnp.bfloat16),
            pltpu.VMEM((N_DEV - 1, CHUNK, HALF), jnp.bfloat16),
            pltpu.SemaphoreType.DMA((N_STEPS, K)),
            pltpu.SemaphoreType.DMA((N_STEPS, K)),
            pltpu.SemaphoreType.DMA((N_STEPS, K)),
            pltpu.SemaphoreType.DMA((N_STEPS, K)),
        ],
        compiler_params=pltpu.CompilerParams(
            collective_id=0,
            vmem_limit_bytes=100 * 1024 * 1024,
        ),
    )(x)


# device time: 169456 ns/iter; 1.8628x vs baseline; 1.0062x over previous
import os

import jax
import jax.numpy as jnp
from jax import lax
from jax.experimental import pallas as pl
from jax.experimental.pallas import tpu as pltpu

_MODE = os.environ.get("KMODE", "full")

N_DEV = 4
M = 4096
N = 2048
CHUNK = M // N_DEV
HALF = N // 2
K = 2
QR = CHUNK // K
N_STEPS = 2 * (N_DEV - 1)


def kernel(x):

    def body(x_ref, out_ref, comm_r, comm_l,
             send_r, recv_r, send_l, recv_l):
        my = lax.axis_index("i")
        right = lax.rem(my + 1, N_DEV)
        left = lax.rem(my + N_DEV - 1, N_DEV)

        barrier_sem = pltpu.get_barrier_semaphore()
        for nbr in (left, right):
            pl.semaphore_signal(
                barrier_sem, inc=1,
                device_id=(nbr,), device_id_type=pl.DeviceIdType.MESH,
            )
        pl.semaphore_wait(barrier_sem, 2)

        def ring(d):
            return (
                (comm_r, send_r, recv_r, right) if d == 0
                else (comm_l, send_l, recv_l, left)
            )

        def rs_desc(s, k, chunk, d):
            comm, ssem, rsem, dev = ring(d)
            return pltpu.make_async_remote_copy(
                src_ref=out_ref.at[pl.ds(chunk * CHUNK + k * QR, QR),
                                   pl.ds(d * HALF, HALF)],
                dst_ref=comm.at[s, pl.ds(k * QR, QR), :],
                send_sem=ssem.at[s, k],
                recv_sem=rsem.at[s, k],
                device_id=(dev,),
                device_id_type=pl.DeviceIdType.MESH,
            )

        def ag_desc(t, k, chunk, d):
            comm, ssem, rsem, dev = ring(d)
            sl = (pl.ds(chunk * CHUNK + k * QR, QR), pl.ds(d * HALF, HALF))
            return pltpu.make_async_remote_copy(
                src_ref=out_ref.at[sl],
                dst_ref=out_ref.at[sl],
                send_sem=ssem.at[N_DEV - 1 + t, k],
                recv_sem=rsem.at[N_DEV - 1 + t, k],
                device_id=(dev,),
                device_id_type=pl.DeviceIdType.MESH,
            )

        do_comm = _MODE in ("full", "comm")
        do_comp = _MODE in ("full", "compute")

        if do_comp:
            out_ref[pl.ds(my * CHUNK, CHUNK), :] = (
                x_ref[0, pl.ds(my * CHUNK, CHUNK), :].astype(jnp.bfloat16)
            )
        if do_comm:
            for k in range(K):
                for d in (0, 1):
                    rs_desc(0, k, my, d).start()

        for s in range(N_DEV - 1):
            for k in range(K):
                for d in (0, 1):
                    if d == 0:
                        recv_c = lax.rem(my - s - 1 + N_DEV, N_DEV)
                    else:
                        recv_c = lax.rem(my + s + 1, N_DEV)
                    comm = ring(d)[0]
                    if do_comm:
                        rs_desc(s, k, 0, d).wait_recv()
                    if do_comp:
                        rows = pl.ds(recv_c * CHUNK + k * QR, QR)
                        cols = pl.ds(d * HALF, HALF)
                        out_ref[rows, cols] = (
                            x_ref[0, rows, cols].astype(jnp.bfloat16)
                            + comm[s, pl.ds(k * QR, QR), :]
                        )
                    if do_comm:
                        if s < N_DEV - 2:
                            rs_desc(s + 1, k, recv_c, d).start()
                        else:
                            ag_desc(0, k, recv_c, d).start()

        if do_comm:
            for t in range(N_DEV - 1):
                for k in range(K):
                    for d in (0, 1):
                        if d == 0:
                            recv_c = lax.rem(my - t + N_DEV, N_DEV)
                        else:
                            recv_c = lax.rem(my + t, N_DEV)
                        ag_desc(t, k, recv_c, d).wait_recv()
                        if t < N_DEV - 2:
                            ag_desc(t + 1, k, recv_c, d).start()

            for s in range(N_DEV - 1):
                for k in range(K):
                    for d in (0, 1):
                        rs_desc(s, k, 0, d).wait_send()
            for t in range(N_DEV - 1):
                for k in range(K):
                    for d in (0, 1):
                        ag_desc(t, k, 0, d).wait_send()

    return pl.pallas_call(
        body,
        out_shape=jax.ShapeDtypeStruct((M, N), jnp.bfloat16),
        in_specs=[pl.BlockSpec(memory_space=pltpu.VMEM)],
        out_specs=pl.BlockSpec(memory_space=pltpu.VMEM),
        scratch_shapes=[
            pltpu.VMEM((N_DEV - 1, CHUNK, HALF), jnp.bfloat16),
            pltpu.VMEM((N_DEV - 1, CHUNK, HALF), jnp.bfloat16),
            pltpu.SemaphoreType.DMA((N_STEPS, K)),
            pltpu.SemaphoreType.DMA((N_STEPS, K)),
            pltpu.SemaphoreType.DMA((N_STEPS, K)),
            pltpu.SemaphoreType.DMA((N_STEPS, K)),
        ],
        compiler_params=pltpu.CompilerParams(
            collective_id=0,
            vmem_limit_bytes=100 * 1024 * 1024,
        ),
    )(x)
